# Initial kernel scaffold; baseline (speedup 1.0000x reference)
import jax
import jax.numpy as jnp
from jax.experimental import pallas as pl


def kernel(boxes, classification):
    raise NotImplementedError("write your pallas kernel here")


# speedup vs baseline: 5.2063x; 5.2063x over previous
import jax
import jax.numpy as jnp
from jax.experimental import pallas as pl
from jax.experimental.pallas import tpu as pltpu

SCORE_THRESH = 0.01
MAX_DET = 300
NEG = float("-inf")
BIG = 2**30


def _fd_kernel(N, n_chunks, scores_ref, boxes_ref, ob_ref, os_ref, ol_ref, masked_ref):
    lane = jax.lax.broadcasted_iota(jnp.int32, (1, 128), 1)
    iota2d = (jax.lax.broadcasted_iota(jnp.int32, (128, 128), 0) * 128
              + jax.lax.broadcasted_iota(jnp.int32, (128, 128), 1))

    def init_body(i, cm):
        blk = scores_ref[0, pl.ds(i * 128, 128), :]
        mblk = jnp.where(blk > SCORE_THRESH, blk, NEG)
        masked_ref[pl.ds(i * 128, 128), :] = mblk
        mi = jnp.max(mblk)
        return jnp.where(lane == i, mi, cm)

    cm0 = jnp.full((1, 128), NEG, jnp.float32)
    cm = jax.lax.fori_loop(0, n_chunks, init_body, cm0)

    def ext_body(it, cm):
        m = jnp.max(cm)
        ci = jnp.min(jnp.where(cm == m, lane, BIG))
        chunk = masked_ref[pl.ds(ci * 128, 128), :]
        idx = iota2d + ci * (128 * 128)
        fidx = jnp.min(jnp.where(chunk == m, idx, BIG))
        valid = m > SCORE_THRESH
        anchor = fidx % N
        label = jnp.where(valid, fidx // N, -1)

        out_lane = jax.lax.broadcasted_iota(jnp.int32, (1, MAX_DET), 1)
        os_ref[0] = jnp.where(out_lane == it,
                              jnp.where(valid, m, -1.0), os_ref[0])
        ol_ref[0] = jnp.where(out_lane == it, label, ol_ref[0])

        p = anchor * 4
        brow = boxes_ref[0, pl.ds(p // 128, 1), :]
        lbase = p % 128
        coords = [jnp.max(jnp.where(lane == lbase + k, brow, NEG))
                  for k in range(4)]
        riota = jax.lax.broadcasted_iota(jnp.int32, (MAX_DET, 4), 0)
        ciota = jax.lax.broadcasted_iota(jnp.int32, (MAX_DET, 4), 1)
        coordrow = sum(jnp.where(ciota == k, coords[k], 0.0) for k in range(4))
        newbox = jnp.where(valid, coordrow, -1.0)
        ob_ref[0] = jnp.where(riota == it, newbox, ob_ref[0])
        newchunk = jnp.where(idx == fidx, NEG, chunk)
        masked_ref[pl.ds(ci * 128, 128), :] = newchunk
        ncm = jnp.max(newchunk)
        return jnp.where(lane == ci, ncm, cm)

    jax.lax.fori_loop(0, MAX_DET, ext_body, cm)


def kernel(boxes, classification):
    B, N, C = classification.shape
    flat = jnp.transpose(classification, (0, 2, 1)).reshape(B, C * N // 128, 128)
    R = flat.shape[1]
    RP = ((R + 127) // 128) * 128
    scores_cm = jnp.pad(flat, ((0, 0), (0, RP - R), (0, 0)), constant_values=-1.0)
    boxes_r = boxes.reshape(B, N * 4 // 128, 128)
    n_chunks = RP // 128

    import functools
    kfn = functools.partial(_fd_kernel, N, n_chunks)
    out_boxes, out_scores, out_labels = pl.pallas_call(
        kfn,
        grid=(B,),
        in_specs=[
            pl.BlockSpec((1, RP, 128), lambda b: (b, 0, 0)),
            pl.BlockSpec((1, N * 4 // 128, 128), lambda b: (b, 0, 0)),
        ],
        out_specs=[
            pl.BlockSpec((1, MAX_DET, 4), lambda b: (b, 0, 0)),
            pl.BlockSpec((1, 1, MAX_DET), lambda b: (b, 0, 0)),
            pl.BlockSpec((1, 1, MAX_DET), lambda b: (b, 0, 0)),
        ],
        out_shape=[
            jax.ShapeDtypeStruct((B, MAX_DET, 4), jnp.float32),
            jax.ShapeDtypeStruct((B, 1, MAX_DET), jnp.float32),
            jax.ShapeDtypeStruct((B, 1, MAX_DET), jnp.int32),
        ],
        scratch_shapes=[pltpu.VMEM((RP, 128), jnp.float32)],
    )(scores_cm, boxes_r)
    return out_boxes, out_scores.reshape(B, MAX_DET), out_labels.reshape(B, MAX_DET)


# speedup vs baseline: 5.2072x; 1.0002x over previous
import jax
import jax.numpy as jnp
from jax.experimental import pallas as pl
from jax.experimental.pallas import tpu as pltpu

SCORE_THRESH = 0.01
MAX_DET = 300
NEG = float("-inf")
BIG = 2**30


def _fd_kernel(N, n_chunks, scores_ref, boxes_ref, ob_ref, os_ref, ol_ref, masked_ref):
    lane = jax.lax.broadcasted_iota(jnp.int32, (1, 128), 1)
    iota2d = (jax.lax.broadcasted_iota(jnp.int32, (128, 128), 0) * 128
              + jax.lax.broadcasted_iota(jnp.int32, (128, 128), 1))

    def init_body(i, cm):
        blk = scores_ref[0, pl.ds(i * 128, 128), :]
        mblk = jnp.where(blk > SCORE_THRESH, blk, NEG)
        masked_ref[pl.ds(i * 128, 128), :] = mblk
        mi = jnp.max(mblk)
        return jnp.where(lane == i, mi, cm)

    cm0 = jnp.full((1, 128), NEG, jnp.float32)
    cm = jax.lax.fori_loop(0, n_chunks, init_body, cm0)

    def ext_body(it, cm):
        m = jnp.max(cm)
        ci = jnp.min(jnp.where(cm == m, lane, BIG))
        chunk = masked_ref[pl.ds(ci * 128, 128), :]
        idx = iota2d + ci * (128 * 128)
        fidx = jnp.min(jnp.where(chunk == m, idx, BIG))
        valid = m > SCORE_THRESH
        anchor = fidx % N
        label = jnp.where(valid, fidx // N, -1)

        out_lane = jax.lax.broadcasted_iota(jnp.int32, (1, MAX_DET), 1)
        os_ref[0] = jnp.where(out_lane == it,
                              jnp.where(valid, m, -1.0), os_ref[0])
        ol_ref[0] = jnp.where(out_lane == it, label, ol_ref[0])

        p = anchor * 4
        brow = boxes_ref[0, pl.ds(p // 128, 1), :]
        lbase = p % 128
        coords = [jnp.max(jnp.where(lane == lbase + k, brow, NEG))
                  for k in range(4)]
        riota = jax.lax.broadcasted_iota(jnp.int32, (MAX_DET, 4), 0)
        ciota = jax.lax.broadcasted_iota(jnp.int32, (MAX_DET, 4), 1)
        coordrow = sum(jnp.where(ciota == k, coords[k], 0.0) for k in range(4))
        newbox = jnp.where(valid, coordrow, -1.0)
        ob_ref[0] = jnp.where(riota == it, newbox, ob_ref[0])
        newchunk = jnp.where(idx == fidx, NEG, chunk)
        masked_ref[pl.ds(ci * 128, 128), :] = newchunk
        ncm = jnp.max(newchunk)
        return jnp.where(lane == ci, ncm, cm)

    jax.lax.fori_loop(0, MAX_DET, ext_body, cm)


def kernel(boxes, classification):
    B, N, C = classification.shape
    flat = jnp.transpose(classification, (0, 2, 1)).reshape(B, C * N // 128, 128)
    R = flat.shape[1]
    RP = ((R + 127) // 128) * 128
    scores_cm = jnp.pad(flat, ((0, 0), (0, RP - R), (0, 0)), constant_values=-1.0)
    boxes_r = boxes.reshape(B, N * 4 // 128, 128)
    n_chunks = RP // 128

    import functools
    kfn = functools.partial(_fd_kernel, N, n_chunks)
    out_boxes, out_scores, out_labels = pl.pallas_call(
        kfn,
        grid=(B,),
        in_specs=[
            pl.BlockSpec((1, RP, 128), lambda b: (b, 0, 0)),
            pl.BlockSpec((1, N * 4 // 128, 128), lambda b: (b, 0, 0)),
        ],
        out_specs=[
            pl.BlockSpec((1, MAX_DET, 4), lambda b: (b, 0, 0)),
            pl.BlockSpec((1, 1, MAX_DET), lambda b: (b, 0, 0)),
            pl.BlockSpec((1, 1, MAX_DET), lambda b: (b, 0, 0)),
        ],
        out_shape=[
            jax.ShapeDtypeStruct((B, MAX_DET, 4), jnp.float32),
            jax.ShapeDtypeStruct((B, 1, MAX_DET), jnp.float32),
            jax.ShapeDtypeStruct((B, 1, MAX_DET), jnp.int32),
        ],
        scratch_shapes=[pltpu.VMEM((RP, 128), jnp.float32)],
        compiler_params=pltpu.CompilerParams(
            dimension_semantics=("parallel",)),
    )(scores_cm, boxes_r)
    return out_boxes, out_scores.reshape(B, MAX_DET), out_labels.reshape(B, MAX_DET)


# speedup vs baseline: 5.3950x; 1.0361x over previous
import functools

import jax
import jax.numpy as jnp
from jax.experimental import pallas as pl
from jax.experimental.pallas import tpu as pltpu

SCORE_THRESH = 0.01
MAX_DET = 300
NEG = float("-inf")
BIG = 2**30
PAIR = 2


def _fd_kernel(N, n_chunks, scores_ref, boxes_ref, ob_ref, os_ref, ol_ref, masked_ref):
    lane = jax.lax.broadcasted_iota(jnp.int32, (1, 128), 1)
    iota2d = (jax.lax.broadcasted_iota(jnp.int32, (128, 128), 0) * 128
              + jax.lax.broadcasted_iota(jnp.int32, (128, 128), 1))

    def init_one(j, i, cm):
        blk = scores_ref[0, j, pl.ds(i * 128, 128), :]
        mblk = jnp.where(blk > SCORE_THRESH, blk, NEG)
        masked_ref[j, pl.ds(i * 128, 128), :] = mblk
        mi = jnp.max(mblk)
        return jnp.where(lane == i, mi, cm)

    def init_body(i, cms):
        return tuple(init_one(j, i, cms[j]) for j in range(PAIR))

    cm0 = jnp.full((1, 128), NEG, jnp.float32)
    cms = jax.lax.fori_loop(0, n_chunks, init_body, (cm0,) * PAIR)

    def ext_one(j, it, cm):
        m = jnp.max(cm)
        ci = jnp.min(jnp.where(cm == m, lane, BIG))
        chunk = masked_ref[j, pl.ds(ci * 128, 128), :]
        idx = iota2d + ci * (128 * 128)
        fidx = jnp.min(jnp.where(chunk == m, idx, BIG))
        valid = m > SCORE_THRESH
        anchor = fidx % N
        label = jnp.where(valid, fidx // N, -1)

        out_lane = jax.lax.broadcasted_iota(jnp.int32, (1, MAX_DET), 1)
        os_ref[0, j] = jnp.where(out_lane == it,
                                 jnp.where(valid, m, -1.0), os_ref[0, j])
        ol_ref[0, j] = jnp.where(out_lane == it, label, ol_ref[0, j])

        p = anchor * 4
        brow = boxes_ref[0, j, pl.ds(p // 128, 1), :]
        lbase = p % 128
        coords = [jnp.max(jnp.where(lane == lbase + k, brow, NEG))
                  for k in range(4)]
        riota = jax.lax.broadcasted_iota(jnp.int32, (MAX_DET, 4), 0)
        ciota = jax.lax.broadcasted_iota(jnp.int32, (MAX_DET, 4), 1)
        coordrow = sum(jnp.where(ciota == k, coords[k], 0.0) for k in range(4))
        newbox = jnp.where(valid, coordrow, -1.0)
        ob_ref[0, j] = jnp.where(riota == it, newbox, ob_ref[0, j])

        newchunk = jnp.where(idx == fidx, NEG, chunk)
        masked_ref[j, pl.ds(ci * 128, 128), :] = newchunk
        ncm = jnp.max(newchunk)
        return jnp.where(lane == ci, ncm, cm)

    def ext_body(it, cms):
        return tuple(ext_one(j, it, cms[j]) for j in range(PAIR))

    jax.lax.fori_loop(0, MAX_DET, ext_body, cms)


def kernel(boxes, classification):
    B, N, C = classification.shape
    flat = jnp.transpose(classification, (0, 2, 1)).reshape(B, C * N // 128, 128)
    R = flat.shape[1]
    RP = ((R + 127) // 128) * 128
    scores_cm = jnp.pad(flat, ((0, 0), (0, RP - R), (0, 0)), constant_values=-1.0)
    G = B // PAIR
    scores_cm = scores_cm.reshape(G, PAIR, RP, 128)
    boxes_r = boxes.reshape(G, PAIR, N * 4 // 128, 128)
    n_chunks = RP // 128
    BR = N * 4 // 128

    kfn = functools.partial(_fd_kernel, N, n_chunks)
    out_boxes, out_scores, out_labels = pl.pallas_call(
        kfn,
        grid=(G,),
        in_specs=[
            pl.BlockSpec((1, PAIR, RP, 128), lambda b: (b, 0, 0, 0)),
            pl.BlockSpec((1, PAIR, BR, 128), lambda b: (b, 0, 0, 0)),
        ],
        out_specs=[
            pl.BlockSpec((1, PAIR, MAX_DET, 4), lambda b: (b, 0, 0, 0)),
            pl.BlockSpec((1, PAIR, 1, MAX_DET), lambda b: (b, 0, 0, 0)),
            pl.BlockSpec((1, PAIR, 1, MAX_DET), lambda b: (b, 0, 0, 0)),
        ],
        out_shape=[
            jax.ShapeDtypeStruct((G, PAIR, MAX_DET, 4), jnp.float32),
            jax.ShapeDtypeStruct((G, PAIR, 1, MAX_DET), jnp.float32),
            jax.ShapeDtypeStruct((G, PAIR, 1, MAX_DET), jnp.int32),
        ],
        scratch_shapes=[pltpu.VMEM((PAIR, RP, 128), jnp.float32)],
        compiler_params=pltpu.CompilerParams(
            dimension_semantics=("parallel",)),
    )(scores_cm, boxes_r)
    return (out_boxes.reshape(B, MAX_DET, 4),
            out_scores.reshape(B, MAX_DET),
            out_labels.reshape(B, MAX_DET))
